# baseline (speedup 1.0000x reference)
import functools

import jax
import jax.numpy as jnp
from jax import lax
from jax.experimental import pallas as pl
from jax.experimental.pallas import tpu as pltpu
from jax.experimental.pallas import tpu_sc as plsc

NC = 2
NS = 16
LANES = 16
SC_SEGS = 2


def _sc_seg_max_body(d, seg_len, first_seg,
                     seqs_hbm, out_hbm, buf, part, sem0, sem1):
    nvec = d // LANES
    c = lax.axis_index("c")
    s = lax.axis_index("s")
    rows_w = seg_len // NS

    row0 = (first_seg + c) * seg_len + s * rows_w
    half = rows_w // 2
    neg_inf = jnp.full((LANES,), -jnp.inf, dtype=jnp.float32)
    accs = (neg_inf,) * nvec

    d1 = pltpu.async_copy(seqs_hbm.at[pl.ds(row0, half)], buf.at[0], sem0)
    d2 = pltpu.async_copy(
        seqs_hbm.at[pl.ds(row0 + half, half)], buf.at[1], sem1)
    for h, dsc in ((0, d1), (1, d2)):
        dsc.wait()

        def row_step(r, a, _h=h):
            return tuple(
                jnp.maximum(a[f], buf[_h, r, pl.ds(f * LANES, LANES)])
                for f in range(nvec)
            )

        accs = lax.fori_loop(0, half, row_step, accs)
    for f in range(nvec):
        part[0, pl.ds(f * LANES, LANES)] = accs[f]
    pltpu.sync_copy(part, out_hbm.at[pl.ds(c * NS + s, 1)])


def _sc_part(seqs2d, d, seg_len, first_seg):
    mesh = plsc.VectorSubcoreMesh(core_axis_name="c", subcore_axis_name="s")
    body = functools.partial(_sc_seg_max_body, d, seg_len, first_seg)
    return pl.kernel(
        body,
        out_type=jax.ShapeDtypeStruct((SC_SEGS * NS, d), jnp.float32),
        mesh=mesh,
        scratch_types=[
            pltpu.VMEM((2, seg_len // NS // 2, d), jnp.float32),
            pltpu.VMEM((1, d), jnp.float32),
            pltpu.SemaphoreType.DMA,
            pltpu.SemaphoreType.DMA,
        ],
    )(seqs2d)


def _tc_seg_max_body(seqs_ref, out_ref):
    i = pl.program_id(0)
    out_ref[pl.ds(i, 1), :] = jnp.max(seqs_ref[...], axis=0, keepdims=True)


def _tc_part(seqs2d, d, seg_len, num_tc_segs):
    return pl.pallas_call(
        _tc_seg_max_body,
        grid=(num_tc_segs,),
        in_specs=[pl.BlockSpec((seg_len, d), lambda i: (i, 0))],
        out_specs=pl.BlockSpec((num_tc_segs, d), lambda i: (0, 0)),
        out_shape=jax.ShapeDtypeStruct((num_tc_segs, d), jnp.float32),
    )(seqs2d)


def kernel(seqs, seqL):
    n, total_s, d = seqs.shape
    num_segs = seqL.shape[1]
    del seqL
    seg_len = total_s // num_segs
    num_tc_segs = num_segs - SC_SEGS

    seqs2d = seqs.reshape(total_s, d)

    sc_parts = _sc_part(seqs2d, d, seg_len, num_tc_segs)
    tc_out = _tc_part(seqs2d, d, seg_len, num_tc_segs)
    sc_out = jnp.max(sc_parts.reshape(SC_SEGS, NS, d), axis=1)
    return jnp.concatenate([tc_out, sc_out], axis=0)


# speedup vs baseline: 1.0411x; 1.0411x over previous
import functools

import jax
import jax.numpy as jnp
from jax import lax
from jax.experimental import pallas as pl
from jax.experimental.pallas import tpu as pltpu
from jax.experimental.pallas import tpu_sc as plsc

NC = 2
NS = 16
LANES = 16
SC_SEGS = 2


def _sc_seg_max_body(d, seg_len, first_seg,
                     seqs_hbm, out_hbm, buf, part, sem0, sem1):
    nvec = d // LANES
    c = lax.axis_index("c")
    s = lax.axis_index("s")
    rows_w = seg_len // NS

    row0 = (first_seg + c) * seg_len + s * rows_w
    half = rows_w // 2
    neg_inf = jnp.full((LANES,), -jnp.inf, dtype=jnp.float32)
    accs = (neg_inf,) * nvec

    d1 = pltpu.async_copy(seqs_hbm.at[pl.ds(row0, half)], buf.at[0], sem0)
    d2 = pltpu.async_copy(
        seqs_hbm.at[pl.ds(row0 + half, half)], buf.at[1], sem1)
    for h, dsc in ((0, d1), (1, d2)):
        dsc.wait()

        def row_step(r, a, _h=h):
            return tuple(
                jnp.maximum(a[f], buf[_h, r, pl.ds(f * LANES, LANES)])
                for f in range(nvec)
            )

        accs = lax.fori_loop(0, half, row_step, accs)
    for f in range(nvec):
        part[0, pl.ds(f * LANES, LANES)] = accs[f]
    pltpu.sync_copy(part, out_hbm.at[pl.ds(c * NS + s, 1)])


def _sc_part(seqs2d, d, seg_len, first_seg):
    mesh = plsc.VectorSubcoreMesh(core_axis_name="c", subcore_axis_name="s")
    body = functools.partial(_sc_seg_max_body, d, seg_len, first_seg)
    return pl.kernel(
        body,
        out_type=jax.ShapeDtypeStruct((SC_SEGS * NS, d), jnp.float32),
        mesh=mesh,
        scratch_types=[
            pltpu.VMEM((2, seg_len // NS // 2, d), jnp.float32),
            pltpu.VMEM((1, d), jnp.float32),
            pltpu.SemaphoreType.DMA,
            pltpu.SemaphoreType.DMA,
        ],
    )(seqs2d)


def _tc_seg_max_body(seqs_ref, out_ref):
    i = pl.program_id(0)
    out_ref[pl.ds(i, 1), :] = jnp.max(seqs_ref[...], axis=0, keepdims=True)


def _tc_part(seqs2d, d, seg_len, num_tc_segs):
    return pl.pallas_call(
        _tc_seg_max_body,
        grid=(num_tc_segs,),
        in_specs=[pl.BlockSpec((seg_len, d), lambda i: (i, 0))],
        out_specs=pl.BlockSpec((num_tc_segs, d), lambda i: (0, 0)),
        out_shape=jax.ShapeDtypeStruct((num_tc_segs, d), jnp.float32),
    )(seqs2d)


def _merge_body(num_tc_segs, tc_ref, sc_ref, out_ref):
    out_ref[pl.ds(0, num_tc_segs), :] = tc_ref[...]
    for k in range(SC_SEGS):
        out_ref[pl.ds(num_tc_segs + k, 1), :] = jnp.max(
            sc_ref[pl.ds(k * NS, NS), :], axis=0, keepdims=True)


def _merge(tc_out, sc_parts, d, num_segs, num_tc_segs):
    return pl.pallas_call(
        functools.partial(_merge_body, num_tc_segs),
        out_shape=jax.ShapeDtypeStruct((num_segs, d), jnp.float32),
    )(tc_out, sc_parts)


def kernel(seqs, seqL):
    n, total_s, d = seqs.shape
    num_segs = seqL.shape[1]
    del seqL
    seg_len = total_s // num_segs
    num_tc_segs = num_segs - SC_SEGS

    seqs2d = seqs.reshape(total_s, d)

    sc_parts = _sc_part(seqs2d, d, seg_len, num_tc_segs)
    tc_out = _tc_part(seqs2d, d, seg_len, num_tc_segs)
    return _merge(tc_out, sc_parts, d, num_segs, num_tc_segs)
